# baseline (speedup 1.0000x reference)
import functools

import jax
import jax.numpy as jnp
from jax import lax
from jax.experimental import pallas as pl
from jax.experimental.pallas import tpu as pltpu
from jax.experimental.pallas import tpu_sc as plsc

VOCAB = 1000000
HIDDEN = 64
ROWS = 4096
COLS = 200
B_TOTAL = ROWS * COLS
NUM_WORKERS = 32
B_PER_W = B_TOTAL // NUM_WORKERS
CHUNK = 1024
N_CHUNKS = B_PER_W // CHUNK
STREAM = 128
K = CHUNK // STREAM


@functools.partial(
    pl.kernel,
    out_type=jax.ShapeDtypeStruct((B_TOTAL, HIDDEN), jnp.float32),
    mesh=plsc.VectorSubcoreMesh(core_axis_name="c", subcore_axis_name="s"),
    compiler_params=pltpu.CompilerParams(use_tc_tiling_on_sc=False),
    scratch_types=[
        pltpu.VMEM((CHUNK,), jnp.int32),
        pltpu.VMEM((CHUNK, HIDDEN), jnp.float32),
        pltpu.SemaphoreType.DMA,
    ],
)
def _emb_lookup(idx_hbm, table_hbm, out_hbm, idx_v, rows_v, sem):
    wid = lax.axis_index("s") * 2 + lax.axis_index("c")
    base = wid * B_PER_W

    def body(c, carry):
        cb = base + c * CHUNK
        pltpu.sync_copy(idx_hbm.at[pl.ds(cb, CHUNK)], idx_v)
        copies = []
        for j in range(K):
            copies.append(
                pltpu.async_copy(
                    table_hbm.at[idx_v.at[pl.ds(j * STREAM, STREAM)]],
                    rows_v.at[pl.ds(j * STREAM, STREAM)],
                    sem,
                )
            )
        for cp in copies:
            cp.wait()
        pltpu.sync_copy(rows_v, out_hbm.at[pl.ds(cb, CHUNK)])
        return carry

    lax.fori_loop(0, N_CHUNKS, body, 0)


def kernel(input_ids, emb_weight):
    flat_idx = input_ids.reshape(-1).astype(jnp.int32)
    out = _emb_lookup(flat_idx, emb_weight)
    return out.reshape(ROWS, COLS, HIDDEN)


# speedup vs baseline: 1.0148x; 1.0148x over previous
import functools

import jax
import jax.numpy as jnp
from jax import lax
from jax.experimental import pallas as pl
from jax.experimental.pallas import tpu as pltpu
from jax.experimental.pallas import tpu_sc as plsc

VOCAB = 1000000
HIDDEN = 64
ROWS = 4096
COLS = 200
B_TOTAL = ROWS * COLS
NUM_WORKERS = 32
B_PER_W = B_TOTAL // NUM_WORKERS
CHUNK = 512
N_CHUNKS = B_PER_W // CHUNK
N_PAIR = N_CHUNKS // 2
STREAM = 128
K = CHUNK // STREAM


@functools.partial(
    pl.kernel,
    out_type=jax.ShapeDtypeStruct((B_TOTAL, HIDDEN), jnp.float32),
    mesh=plsc.VectorSubcoreMesh(core_axis_name="c", subcore_axis_name="s"),
    compiler_params=pltpu.CompilerParams(use_tc_tiling_on_sc=False),
    scratch_types=[
        pltpu.VMEM((B_PER_W,), jnp.int32),
        pltpu.VMEM((2, CHUNK, HIDDEN), jnp.float32),
        pltpu.SemaphoreType.DMA,
        pltpu.SemaphoreType.DMA,
    ],
)
def _emb_lookup(idx_hbm, table_hbm, out_hbm, idx_all, rows, sem_g, sem_o):
    wid = lax.axis_index("s") * 2 + lax.axis_index("c")
    base = wid * B_PER_W
    pltpu.sync_copy(idx_hbm.at[pl.ds(base, B_PER_W)], idx_all)

    def pair(i, carry):
        for b in range(2):
            c = i * 2 + b

            @pl.when(i > 0)
            def _():
                pltpu.make_async_copy(
                    rows.at[b], out_hbm.at[pl.ds(base, CHUNK)], sem_o
                ).wait()

            gathers = [
                pltpu.async_copy(
                    table_hbm.at[idx_all.at[pl.ds(c * CHUNK + j * STREAM, STREAM)]],
                    rows.at[b, pl.ds(j * STREAM, STREAM)],
                    sem_g,
                )
                for j in range(K)
            ]
            for g in gathers:
                g.wait()
            pltpu.async_copy(
                rows.at[b], out_hbm.at[pl.ds(base + c * CHUNK, CHUNK)], sem_o
            )
        return carry

    lax.fori_loop(0, N_PAIR, pair, 0)
    for b in range(2):
        pltpu.make_async_copy(
            rows.at[b], out_hbm.at[pl.ds(base, CHUNK)], sem_o
        ).wait()


def kernel(input_ids, emb_weight):
    flat_idx = input_ids.reshape(-1).astype(jnp.int32)
    out = _emb_lookup(flat_idx, emb_weight)
    return out.reshape(ROWS, COLS, HIDDEN)
